# baseline (speedup 1.0000x reference)
import functools

import jax
import jax.numpy as jnp
from jax.experimental import pallas as pl
from jax.experimental.pallas import tpu as pltpu

STEP_SIZE = 0.1
MAX_RESAMPLINGS = 50
N_ROWS = 16384
N_COLS = 128
BLOCK_ROWS = 256


def _threefry2x32(k0, k1, x0, x1):
    def rotl(v, r):
        return (v << jnp.uint32(r)) | (v >> jnp.uint32(32 - r))

    ks2 = k0 ^ k1 ^ jnp.uint32(0x1BD11BDA)
    rots = ((13, 15, 26, 6), (17, 29, 16, 24))
    inject = ((k1, ks2, 1), (ks2, k0, 2), (k0, k1, 3), (k1, ks2, 4), (ks2, k0, 5))
    x0 = x0 + k0
    x1 = x1 + k1
    for g in range(5):
        for r in rots[g % 2]:
            x0 = x0 + x1
            x1 = rotl(x1, r)
            x1 = x0 ^ x1
        a, b, c = inject[g]
        x0 = x0 + a
        x1 = x1 + b + jnp.uint32(c)
    return x0, x1


def _uniform(k0, k1, idx):
    o0, o1 = _threefry2x32(k0, k1, jnp.zeros_like(idx), idx)
    bits = o0 ^ o1
    f = jax.lax.bitcast_convert_type(
        (bits >> jnp.uint32(9)) | jnp.uint32(0x3F800000), jnp.float32
    )
    return f - 1.0


SUB_ROWS = 8


def _body(kd_ref, x_ref, out_ref):
    i_blk = pl.program_id(0)
    base_row = i_blk * BLOCK_ROWS

    col_ids = jax.lax.broadcasted_iota(jnp.uint32, (SUB_ROWS, N_COLS), 1)
    sub_iota = jax.lax.broadcasted_iota(jnp.uint32, (SUB_ROWS, 1), 0)

    def key_pair(k):
        return kd_ref[k, 0], kd_ref[k, 1]

    def sub_block(s, carry_unused):
        r0 = s * SUB_ROWS
        x = x_ref[pl.ds(r0, SUB_ROWS), :]
        row_ids = (base_row + r0).astype(jnp.uint32) + sub_iota
        flat_ids = row_ids * jnp.uint32(N_COLS) + col_ids

        pot_x = -0.5 * jnp.sum(x * x, axis=-1, keepdims=True)

        k0, k1 = key_pair(0)
        y = jnp.log(1.0 - _uniform(k0, k1, row_ids)) + pot_x

        k0, k1 = key_pair(2)
        sw = jnp.log(1.0 - _uniform(k0, k1, flat_ids)) / (-STEP_SIZE)
        a = x - sw / 2.0
        b = x + sw / 2.0

        k0, k1 = key_pair(3)
        x_new = _uniform(k0, k1, flat_ids) * (b - a) + a
        pot = -0.5 * jnp.sum(x_new * x_new, axis=-1, keepdims=True)
        rej_f = jnp.where(pot < y, 1.0, 0.0).astype(jnp.float32)

        def cond(carry):
            i, _, _, _, rej_f = carry
            return jnp.logical_and(i < MAX_RESAMPLINGS, jnp.sum(rej_f) > 0.0)

        def step(carry):
            i, a, b, x_new, rej_f = carry
            reject = rej_f > 0.0
            smaller = x_new < x
            a = jnp.where(reject & smaller, x_new, a)
            b = jnp.where(reject & (~smaller), x_new, b)
            u = _uniform(kd_ref[4 + i, 0], kd_ref[4 + i, 1], flat_ids)
            x_new = jnp.where(reject, u * (b - a) + a, x_new)
            pot = -0.5 * jnp.sum(x_new * x_new, axis=-1, keepdims=True)
            rej_f = jnp.where(reject & (pot < y), 1.0, 0.0).astype(jnp.float32)
            return i + 1, a, b, x_new, rej_f

        _, _, _, x_new, _ = jax.lax.while_loop(
            cond, step, (jnp.int32(0), a, b, x_new, rej_f)
        )
        out_ref[pl.ds(r0, SUB_ROWS), :] = x_new
        return carry_unused

    jax.lax.fori_loop(0, BLOCK_ROWS // SUB_ROWS, sub_block, jnp.int32(0))


@jax.jit
def kernel(x):
    ks = jax.random.split(jax.random.key(1), 4 + MAX_RESAMPLINGS)
    kd = jax.random.key_data(ks).astype(jnp.uint32)
    return pl.pallas_call(
        _body,
        grid=(N_ROWS // BLOCK_ROWS,),
        in_specs=[
            pl.BlockSpec(memory_space=pltpu.SMEM),
            pl.BlockSpec((BLOCK_ROWS, N_COLS), lambda i: (i, 0)),
        ],
        out_specs=pl.BlockSpec((BLOCK_ROWS, N_COLS), lambda i: (i, 0)),
        out_shape=jax.ShapeDtypeStruct((N_ROWS, N_COLS), jnp.float32),
        compiler_params=pltpu.CompilerParams(
            dimension_semantics=("parallel",),
        ),
    )(kd, x)


# speedup vs baseline: 4.8862x; 4.8862x over previous
import functools

import jax
import jax.numpy as jnp
from jax.experimental import pallas as pl
from jax.experimental.pallas import tpu as pltpu

STEP_SIZE = 0.1
MAX_RESAMPLINGS = 50
N_ROWS = 16384
N_COLS = 128
BLOCK_ROWS = 256


def _threefry2x32(k0, k1, x0, x1):
    def rotl(v, r):
        return (v << jnp.uint32(r)) | (v >> jnp.uint32(32 - r))

    ks2 = k0 ^ k1 ^ jnp.uint32(0x1BD11BDA)
    rots = ((13, 15, 26, 6), (17, 29, 16, 24))
    inject = ((k1, ks2, 1), (ks2, k0, 2), (k0, k1, 3), (k1, ks2, 4), (ks2, k0, 5))
    x0 = x0 + k0
    x1 = x1 + k1
    for g in range(5):
        for r in rots[g % 2]:
            x0 = x0 + x1
            x1 = rotl(x1, r)
            x1 = x0 ^ x1
        a, b, c = inject[g]
        x0 = x0 + a
        x1 = x1 + b + jnp.uint32(c)
    return x0, x1


def _uniform(k0, k1, idx):
    o0, o1 = _threefry2x32(k0, k1, jnp.zeros_like(idx), idx)
    bits = o0 ^ o1
    f = jax.lax.bitcast_convert_type(
        (bits >> jnp.uint32(9)) | jnp.uint32(0x3F800000), jnp.float32
    )
    return f - 1.0


SUB_ROWS = 64


def _body(kd_ref, x_ref, out_ref):
    i_blk = pl.program_id(0)
    base_row = i_blk * BLOCK_ROWS

    col_ids = jax.lax.broadcasted_iota(jnp.uint32, (SUB_ROWS, N_COLS), 1)
    sub_iota = jax.lax.broadcasted_iota(jnp.uint32, (SUB_ROWS, 1), 0)

    def key_pair(k):
        return kd_ref[k, 0], kd_ref[k, 1]

    def sub_block(s, carry_unused):
        r0 = s * SUB_ROWS
        x = x_ref[pl.ds(r0, SUB_ROWS), :]
        row_ids = (base_row + r0).astype(jnp.uint32) + sub_iota
        flat_ids = row_ids * jnp.uint32(N_COLS) + col_ids

        pot_x = -0.5 * jnp.sum(x * x, axis=-1, keepdims=True)

        k0, k1 = key_pair(0)
        y = jnp.log(1.0 - _uniform(k0, k1, row_ids)) + pot_x

        k0, k1 = key_pair(2)
        sw = jnp.log(1.0 - _uniform(k0, k1, flat_ids)) / (-STEP_SIZE)
        a = x - sw / 2.0
        b = x + sw / 2.0

        k0, k1 = key_pair(3)
        x_new = _uniform(k0, k1, flat_ids) * (b - a) + a
        pot = -0.5 * jnp.sum(x_new * x_new, axis=-1, keepdims=True)
        rej_f = jnp.where(pot < y, 1.0, 0.0).astype(jnp.float32)

        def cond(carry):
            i, _, _, _, rej_f = carry
            return jnp.logical_and(i < MAX_RESAMPLINGS, jnp.sum(rej_f) > 0.0)

        def step(carry):
            i, a, b, x_new, rej_f = carry
            reject = rej_f > 0.0
            smaller = x_new < x
            a = jnp.where(reject & smaller, x_new, a)
            b = jnp.where(reject & (~smaller), x_new, b)
            u = _uniform(kd_ref[4 + i, 0], kd_ref[4 + i, 1], flat_ids)
            x_new = jnp.where(reject, u * (b - a) + a, x_new)
            pot = -0.5 * jnp.sum(x_new * x_new, axis=-1, keepdims=True)
            rej_f = jnp.where(reject & (pot < y), 1.0, 0.0).astype(jnp.float32)
            return i + 1, a, b, x_new, rej_f

        _, _, _, x_new, _ = jax.lax.while_loop(
            cond, step, (jnp.int32(0), a, b, x_new, rej_f)
        )
        out_ref[pl.ds(r0, SUB_ROWS), :] = x_new
        return carry_unused

    jax.lax.fori_loop(0, BLOCK_ROWS // SUB_ROWS, sub_block, jnp.int32(0))


@jax.jit
def kernel(x):
    ks = jax.random.split(jax.random.key(1), 4 + MAX_RESAMPLINGS)
    kd = jax.random.key_data(ks).astype(jnp.uint32)
    return pl.pallas_call(
        _body,
        grid=(N_ROWS // BLOCK_ROWS,),
        in_specs=[
            pl.BlockSpec(memory_space=pltpu.SMEM),
            pl.BlockSpec((BLOCK_ROWS, N_COLS), lambda i: (i, 0)),
        ],
        out_specs=pl.BlockSpec((BLOCK_ROWS, N_COLS), lambda i: (i, 0)),
        out_shape=jax.ShapeDtypeStruct((N_ROWS, N_COLS), jnp.float32),
        compiler_params=pltpu.CompilerParams(
            dimension_semantics=("parallel",),
        ),
    )(kd, x)


# speedup vs baseline: 8.1682x; 1.6717x over previous
import functools

import jax
import jax.numpy as jnp
from jax.experimental import pallas as pl
from jax.experimental.pallas import tpu as pltpu

STEP_SIZE = 0.1
MAX_RESAMPLINGS = 50
N_ROWS = 16384
N_COLS = 128
BLOCK_ROWS = 512


def _threefry2x32(k0, k1, x0, x1):
    def rotl(v, r):
        return (v << jnp.uint32(r)) | (v >> jnp.uint32(32 - r))

    ks2 = k0 ^ k1 ^ jnp.uint32(0x1BD11BDA)
    rots = ((13, 15, 26, 6), (17, 29, 16, 24))
    inject = ((k1, ks2, 1), (ks2, k0, 2), (k0, k1, 3), (k1, ks2, 4), (ks2, k0, 5))
    x0 = x0 + k0
    x1 = x1 + k1
    for g in range(5):
        for r in rots[g % 2]:
            x0 = x0 + x1
            x1 = rotl(x1, r)
            x1 = x0 ^ x1
        a, b, c = inject[g]
        x0 = x0 + a
        x1 = x1 + b + jnp.uint32(c)
    return x0, x1


def _uniform(k0, k1, idx):
    o0, o1 = _threefry2x32(k0, k1, jnp.zeros_like(idx), idx)
    bits = o0 ^ o1
    f = jax.lax.bitcast_convert_type(
        (bits >> jnp.uint32(9)) | jnp.uint32(0x3F800000), jnp.float32
    )
    return f - 1.0


SUB_ROWS = BLOCK_ROWS


def _body(kd_ref, x_ref, out_ref):
    i_blk = pl.program_id(0)
    base_row = i_blk * BLOCK_ROWS

    col_ids = jax.lax.broadcasted_iota(jnp.uint32, (SUB_ROWS, N_COLS), 1)
    sub_iota = jax.lax.broadcasted_iota(jnp.uint32, (SUB_ROWS, 1), 0)

    def key_pair(k):
        return kd_ref[k, 0], kd_ref[k, 1]

    def sub_block(s, carry_unused):
        r0 = s * SUB_ROWS
        x = x_ref[pl.ds(r0, SUB_ROWS), :]
        row_ids = (base_row + r0).astype(jnp.uint32) + sub_iota
        flat_ids = row_ids * jnp.uint32(N_COLS) + col_ids

        pot_x = -0.5 * jnp.sum(x * x, axis=-1, keepdims=True)

        k0, k1 = key_pair(0)
        y = jnp.log(1.0 - _uniform(k0, k1, row_ids)) + pot_x

        k0, k1 = key_pair(2)
        sw = jnp.log(1.0 - _uniform(k0, k1, flat_ids)) / (-STEP_SIZE)
        a = x - sw / 2.0
        b = x + sw / 2.0

        k0, k1 = key_pair(3)
        x_new = _uniform(k0, k1, flat_ids) * (b - a) + a
        pot = -0.5 * jnp.sum(x_new * x_new, axis=-1, keepdims=True)
        rej_f = jnp.where(pot < y, 1.0, 0.0).astype(jnp.float32)

        def cond(carry):
            i, _, _, _, rej_f = carry
            return jnp.logical_and(i < MAX_RESAMPLINGS, jnp.sum(rej_f) > 0.0)

        def step(carry):
            i, a, b, x_new, rej_f = carry
            reject = rej_f > 0.0
            smaller = x_new < x
            a = jnp.where(reject & smaller, x_new, a)
            b = jnp.where(reject & (~smaller), x_new, b)
            u = _uniform(kd_ref[4 + i, 0], kd_ref[4 + i, 1], flat_ids)
            x_new = jnp.where(reject, u * (b - a) + a, x_new)
            pot = -0.5 * jnp.sum(x_new * x_new, axis=-1, keepdims=True)
            rej_f = jnp.where(reject & (pot < y), 1.0, 0.0).astype(jnp.float32)
            return i + 1, a, b, x_new, rej_f

        _, _, _, x_new, _ = jax.lax.while_loop(
            cond, step, (jnp.int32(0), a, b, x_new, rej_f)
        )
        out_ref[pl.ds(r0, SUB_ROWS), :] = x_new
        return carry_unused

    jax.lax.fori_loop(0, BLOCK_ROWS // SUB_ROWS, sub_block, jnp.int32(0))


@jax.jit
def kernel(x):
    ks = jax.random.split(jax.random.key(1), 4 + MAX_RESAMPLINGS)
    kd = jax.random.key_data(ks).astype(jnp.uint32)
    return pl.pallas_call(
        _body,
        grid=(N_ROWS // BLOCK_ROWS,),
        in_specs=[
            pl.BlockSpec(memory_space=pltpu.SMEM),
            pl.BlockSpec((BLOCK_ROWS, N_COLS), lambda i: (i, 0)),
        ],
        out_specs=pl.BlockSpec((BLOCK_ROWS, N_COLS), lambda i: (i, 0)),
        out_shape=jax.ShapeDtypeStruct((N_ROWS, N_COLS), jnp.float32),
        compiler_params=pltpu.CompilerParams(
            dimension_semantics=("parallel",),
        ),
    )(kd, x)


# speedup vs baseline: 8.2289x; 1.0074x over previous
import functools

import jax
import jax.numpy as jnp
from jax.experimental import pallas as pl
from jax.experimental.pallas import tpu as pltpu

STEP_SIZE = 0.1
MAX_RESAMPLINGS = 50
N_ROWS = 16384
N_COLS = 128
BLOCK_ROWS = 1024


def _threefry2x32(k0, k1, x0, x1):
    def rotl(v, r):
        return (v << jnp.uint32(r)) | (v >> jnp.uint32(32 - r))

    ks2 = k0 ^ k1 ^ jnp.uint32(0x1BD11BDA)
    rots = ((13, 15, 26, 6), (17, 29, 16, 24))
    inject = ((k1, ks2, 1), (ks2, k0, 2), (k0, k1, 3), (k1, ks2, 4), (ks2, k0, 5))
    x0 = x0 + k0
    x1 = x1 + k1
    for g in range(5):
        for r in rots[g % 2]:
            x0 = x0 + x1
            x1 = rotl(x1, r)
            x1 = x0 ^ x1
        a, b, c = inject[g]
        x0 = x0 + a
        x1 = x1 + b + jnp.uint32(c)
    return x0, x1


def _uniform(k0, k1, idx):
    o0, o1 = _threefry2x32(k0, k1, jnp.zeros_like(idx), idx)
    bits = o0 ^ o1
    f = jax.lax.bitcast_convert_type(
        (bits >> jnp.uint32(9)) | jnp.uint32(0x3F800000), jnp.float32
    )
    return f - 1.0


SUB_ROWS = BLOCK_ROWS


def _body(kd_ref, x_ref, out_ref):
    i_blk = pl.program_id(0)
    base_row = i_blk * BLOCK_ROWS

    col_ids = jax.lax.broadcasted_iota(jnp.uint32, (SUB_ROWS, N_COLS), 1)
    sub_iota = jax.lax.broadcasted_iota(jnp.uint32, (SUB_ROWS, 1), 0)

    def key_pair(k):
        return kd_ref[k, 0], kd_ref[k, 1]

    def sub_block(s, carry_unused):
        r0 = s * SUB_ROWS
        x = x_ref[pl.ds(r0, SUB_ROWS), :]
        row_ids = (base_row + r0).astype(jnp.uint32) + sub_iota
        flat_ids = row_ids * jnp.uint32(N_COLS) + col_ids

        pot_x = -0.5 * jnp.sum(x * x, axis=-1, keepdims=True)

        k0, k1 = key_pair(0)
        y = jnp.log(1.0 - _uniform(k0, k1, row_ids)) + pot_x

        k0, k1 = key_pair(2)
        sw = jnp.log(1.0 - _uniform(k0, k1, flat_ids)) / (-STEP_SIZE)
        a = x - sw / 2.0
        b = x + sw / 2.0

        k0, k1 = key_pair(3)
        x_new = _uniform(k0, k1, flat_ids) * (b - a) + a
        pot = -0.5 * jnp.sum(x_new * x_new, axis=-1, keepdims=True)
        rej_f = jnp.where(pot < y, 1.0, 0.0).astype(jnp.float32)

        def cond(carry):
            i, _, _, _, rej_f = carry
            return jnp.logical_and(i < MAX_RESAMPLINGS, jnp.sum(rej_f) > 0.0)

        def step(carry):
            i, a, b, x_new, rej_f = carry
            reject = rej_f > 0.0
            smaller = x_new < x
            a = jnp.where(reject & smaller, x_new, a)
            b = jnp.where(reject & (~smaller), x_new, b)
            u = _uniform(kd_ref[4 + i, 0], kd_ref[4 + i, 1], flat_ids)
            x_new = jnp.where(reject, u * (b - a) + a, x_new)
            pot = -0.5 * jnp.sum(x_new * x_new, axis=-1, keepdims=True)
            rej_f = jnp.where(reject & (pot < y), 1.0, 0.0).astype(jnp.float32)
            return i + 1, a, b, x_new, rej_f

        _, _, _, x_new, _ = jax.lax.while_loop(
            cond, step, (jnp.int32(0), a, b, x_new, rej_f)
        )
        out_ref[pl.ds(r0, SUB_ROWS), :] = x_new
        return carry_unused

    jax.lax.fori_loop(0, BLOCK_ROWS // SUB_ROWS, sub_block, jnp.int32(0))


@jax.jit
def kernel(x):
    ks = jax.random.split(jax.random.key(1), 4 + MAX_RESAMPLINGS)
    kd = jax.random.key_data(ks).astype(jnp.uint32)
    return pl.pallas_call(
        _body,
        grid=(N_ROWS // BLOCK_ROWS,),
        in_specs=[
            pl.BlockSpec(memory_space=pltpu.SMEM),
            pl.BlockSpec((BLOCK_ROWS, N_COLS), lambda i: (i, 0)),
        ],
        out_specs=pl.BlockSpec((BLOCK_ROWS, N_COLS), lambda i: (i, 0)),
        out_shape=jax.ShapeDtypeStruct((N_ROWS, N_COLS), jnp.float32),
        compiler_params=pltpu.CompilerParams(
            dimension_semantics=("parallel",),
        ),
    )(kd, x)
